# baseline (speedup 1.0000x reference)
import functools

import jax
import jax.numpy as jnp
from jax import lax
from jax.experimental import pallas as pl
from jax.experimental.pallas import tpu as pltpu
from jax.experimental.pallas import tpu_sc as plsc

N = 100000
E = 3200000
D = 16
NT = 100352
NTF = NT // 8
NC, NS = 2, 16
LANE = 128
CB = 4
RPT = 784
RPTD = RPT
RT = NC * NS * RPT
EPAD = RT * LANE
SLICE = NT // NS
CBL = CB * LANE
NS0 = 44
NS1 = 54
EC0 = NS0 * 4 * CBL
EC1 = NS1 * 4 * CBL

_mesh = plsc.VectorSubcoreMesh(
    core_axis_name="c", subcore_axis_name="s", num_cores=NC, num_subcores=NS
)


@functools.partial(
    pl.kernel,
    out_type=jax.ShapeDtypeStruct((NC, 2, NT), jnp.float32),
    mesh=_mesh,
    compiler_params=pltpu.CompilerParams(use_tc_tiling_on_sc=False),
    scratch_types=[
        pltpu.VMEM((4, CBL), jnp.int32),
        pltpu.VMEM((4, CBL), jnp.int32),
        pltpu.VMEM((CBL,), jnp.float32),
        pltpu.VMEM((SLICE,), jnp.float32),
        pltpu.VMEM_SHARED((NT,), jnp.float32),
        pltpu.VMEM_SHARED((NT,), jnp.float32),
        pltpu.SemaphoreType.DMA,
        pltpu.SemaphoreType.DMA,
        pltpu.SemaphoreType.DMA,
        pltpu.SemaphoreType.DMA,
        pltpu.SemaphoreType.DMA,
        pltpu.SemaphoreType.DMA,
        pltpu.SemaphoreType.DMA,
        pltpu.SemaphoreType.DMA,
    ],
)
def _deg_kernel(src_hbm, dst_hbm, out_hbm, src_v, dst_v, ones_v, z_v,
                dego_s, degi_s, is0, is1, is2, is3, ss0, ss1, ss2, ss3):
    c = lax.axis_index("c")
    s = lax.axis_index("s")
    wid = c * NS + s
    isem = (is0, is1, is2, is3)
    ssem = (ss0, ss1, ss2, ss3)

    for i in range(CBL // 16):
        ones_v[pl.ds(i * 16, 16)] = jnp.ones((16,), jnp.float32)

    def zfill(i, _):
        z_v[pl.ds(i * 16, 16)] = jnp.zeros((16,), jnp.float32)
        return 0

    lax.fori_loop(0, SLICE // 16, zfill, 0)
    pltpu.sync_copy(z_v, dego_s.at[pl.ds(s * SLICE, SLICE)])
    pltpu.sync_copy(z_v, degi_s.at[pl.ds(s * SLICE, SLICE)])
    plsc.subcore_barrier()

    base = jnp.where(c == 0, s * EC0, NS * EC0 + s * EC1)
    nsup = jnp.where(c == 0, NS0, NS1)
    nit = 4 * nsup

    def load_idx(slot, off):
        pltpu.async_copy(src_hbm.at[pl.ds(off, CBL)], src_v.at[slot],
                         isem[slot])
        pltpu.async_copy(dst_hbm.at[pl.ds(off, CBL)], dst_v.at[slot],
                         isem[slot])

    def wait_idx(slot):
        pltpu.make_async_copy(src_hbm.at[pl.ds(base, CBL)], src_v.at[slot],
                              isem[slot]).wait()
        pltpu.make_async_copy(dst_hbm.at[pl.ds(base, CBL)], dst_v.at[slot],
                              isem[slot]).wait()

    def fire_scatters(slot):
        pltpu.async_copy(ones_v, dego_s.at[src_v.at[slot]],
                         ssem[slot], add=True)
        pltpu.async_copy(ones_v, degi_s.at[dst_v.at[slot]],
                         ssem[slot], add=True)

    def wait_scatters(slot):
        pltpu.make_async_copy(ones_v, dego_s.at[src_v.at[slot]],
                              ssem[slot]).wait()
        pltpu.make_async_copy(ones_v, degi_s.at[dst_v.at[slot]],
                              ssem[slot]).wait()

    load_idx(0, base)
    load_idx(1, base + CBL)

    def super_body(sp, _):
        for b in range(4):
            s2 = (b + 2) % 4
            it = 4 * sp + b
            wait_idx(b)
            fire_scatters(b)
            if b < 2:
                @pl.when(jnp.logical_and(it + 2 < nit, sp > 0))
                def _():
                    wait_scatters(s2)
                    load_idx(s2, base + (it + 2) * CBL)

                @pl.when(jnp.logical_and(it + 2 < nit, sp == 0))
                def _():
                    load_idx(s2, base + (it + 2) * CBL)
            else:
                @pl.when(it + 2 < nit)
                def _():
                    wait_scatters(s2)
                    load_idx(s2, base + (it + 2) * CBL)
        return 0

    lax.fori_loop(0, nsup, super_body, 0)

    for b in range(4):
        wait_scatters(b)

    plsc.subcore_barrier()
    pltpu.sync_copy(dego_s.at[pl.ds(s * SLICE, SLICE)],
                    out_hbm.at[c, 0, pl.ds(s * SLICE, SLICE)])
    pltpu.sync_copy(degi_s.at[pl.ds(s * SLICE, SLICE)],
                    out_hbm.at[c, 1, pl.ds(s * SLICE, SLICE)])


@functools.partial(
    pl.kernel,
    out_type=jax.ShapeDtypeStruct((NC, NT, D), jnp.float32),
    mesh=_mesh,
    compiler_params=pltpu.CompilerParams(use_tc_tiling_on_sc=False),
    scratch_types=[
        pltpu.VMEM((4, CBL), jnp.int32),
        pltpu.VMEM((4, CBL), jnp.int32),
        pltpu.VMEM((2, CBL, D), jnp.float32),
        pltpu.VMEM((LANE, D), jnp.float32),
        pltpu.VMEM_SHARED((NT, D), jnp.float32),
        pltpu.SemaphoreType.DMA,
        pltpu.SemaphoreType.DMA,
        pltpu.SemaphoreType.DMA,
        pltpu.SemaphoreType.DMA,
        pltpu.SemaphoreType.DMA,
        pltpu.SemaphoreType.DMA,
        pltpu.SemaphoreType.DMA,
        pltpu.SemaphoreType.DMA,
    ],
)
def _seg_kernel(g_hbm, src_hbm, dst_hbm, out_hbm, src_v, dst_v, rows_v,
                zrow_v, acc_s, gsem0, gsem1, ssem0, ssem1,
                isem0, isem1, isem2, isem3):
    c = lax.axis_index("c")
    s = lax.axis_index("s")
    wid = c * NS + s
    gsem = (gsem0, gsem1)
    ssem = (ssem0, ssem1)
    isem = (isem0, isem1, isem2, isem3)

    for i in range(LANE):
        zrow_v[i, :] = jnp.zeros((D,), jnp.float32)

    def zfill(k, _):
        pltpu.async_copy(zrow_v,
                         acc_s.at[pl.ds(s * SLICE + k * LANE, LANE)],
                         gsem0)
        return 0

    lax.fori_loop(0, SLICE // LANE, zfill, 0)

    def zdrain(k, _):
        pltpu.make_async_copy(
            zrow_v, acc_s.at[pl.ds(s * SLICE + k * LANE, LANE)],
            gsem0).wait()
        return 0

    lax.fori_loop(0, SLICE // LANE, zdrain, 0)
    plsc.subcore_barrier()

    base = jnp.where(c == 0, s * EC0, NS * EC0 + s * EC1)
    nsup = jnp.where(c == 0, NS0, NS1)

    def load_idx(slot, off):
        pltpu.async_copy(src_hbm.at[pl.ds(off, CBL)], src_v.at[slot],
                         isem[slot])
        pltpu.async_copy(dst_hbm.at[pl.ds(off, CBL)], dst_v.at[slot],
                         isem[slot])

    def wait_idx(slot):
        pltpu.make_async_copy(src_hbm.at[pl.ds(base, CBL)], src_v.at[slot],
                              isem[slot]).wait()
        pltpu.make_async_copy(dst_hbm.at[pl.ds(base, CBL)], dst_v.at[slot],
                              isem[slot]).wait()

    def fire_gathers(rs, qs):
        pltpu.async_copy(g_hbm.at[src_v.at[qs]], rows_v.at[rs], gsem[rs])

    def wait_gathers(rs, qs):
        pltpu.make_async_copy(g_hbm.at[src_v.at[qs]], rows_v.at[rs],
                              gsem[rs]).wait()

    def fire_scatters(rs, qs):
        pltpu.async_copy(rows_v.at[rs], acc_s.at[dst_v.at[qs]], ssem[rs],
                         add=True)

    def wait_scatters(rs, qs):
        pltpu.make_async_copy(rows_v.at[rs], acc_s.at[dst_v.at[qs]],
                              ssem[rs]).wait()

    load_idx(0, base)
    load_idx(1, base + CBL)

    def super_body(sp, _):
        for u in range(4):
            b = u % 2
            q = u
            q2 = (u + 2) % 4
            ob = 1 - b
            oq = (u + 3) % 4
            j = 4 * sp + u
            if u < 2:
                @pl.when(sp > 0)
                def _():
                    wait_scatters(b, q2)
            else:
                wait_scatters(b, q2)
            if u < 2:
                load_idx(q2, base + (j + 2) * CBL)
            else:
                @pl.when(sp < nsup - 1)
                def _():
                    load_idx(q2, base + (j + 2) * CBL)
            wait_idx(q)
            fire_gathers(b, q)
            if u == 0:
                @pl.when(sp > 0)
                def _():
                    wait_gathers(ob, oq)
                    fire_scatters(ob, oq)
            else:
                wait_gathers(ob, oq)
                fire_scatters(ob, oq)
        return 0

    lax.fori_loop(0, nsup, super_body, 0)

    wait_gathers(1, 3)
    fire_scatters(1, 3)
    wait_scatters(0, 2)
    wait_scatters(1, 3)

    plsc.subcore_barrier()
    pltpu.sync_copy(acc_s.at[pl.ds(s * SLICE, SLICE)],
                    out_hbm.at[c, pl.ds(s * SLICE, SLICE)])


NDF = NT // 128
DBLK = 112
BF = 1568


def _norms_body(d00_ref, d01_ref, d10_ref, d11_ref, ns_ref, nd_ref, ns2_ref):
    i = pl.program_id(0)
    dego = d00_ref[...] + d10_ref[...]
    degi = d01_ref[...] + d11_ref[...]
    ns = lax.rsqrt(jnp.maximum(dego, 1.0))
    nd = lax.rsqrt(jnp.maximum(degi, 1.0))
    node = (i * (DBLK * 128)
            + lax.broadcasted_iota(jnp.int32, (DBLK, 128), 0) * 128
            + lax.broadcasted_iota(jnp.int32, (DBLK, 128), 1))
    valid = node < N
    ns_ref[...] = jnp.where(valid, ns, 0.0)
    nd_ref[...] = jnp.where(valid, nd, 0.0)
    ns2_ref[...] = jnp.where(valid, ns * nd, 0.0)


_NDG = NDF // DBLK

_norms_call = pl.pallas_call(
    _norms_body,
    grid=(_NDG,),
    in_specs=[
        pl.BlockSpec((DBLK, 128), lambda i: (i, 0)),
        pl.BlockSpec((DBLK, 128), lambda i: (_NDG + i, 0)),
        pl.BlockSpec((DBLK, 128), lambda i: (2 * _NDG + i, 0)),
        pl.BlockSpec((DBLK, 128), lambda i: (3 * _NDG + i, 0)),
    ],
    out_specs=[pl.BlockSpec((DBLK, 128), lambda i: (i, 0))] * 3,
    out_shape=[jax.ShapeDtypeStruct((NDF, 128), jnp.float32)] * 3,
)


def _g0_body(x_ref, nse_ref, wp_ref, g_ref):
    g_ref[...] = jnp.dot(x_ref[...] * nse_ref[...], wp_ref[...],
                         preferred_element_type=jnp.float32)


_g0_call = pl.pallas_call(
    _g0_body,
    grid=(NTF // BF,),
    in_specs=[
        pl.BlockSpec((BF, 128), lambda i: (i, 0)),
        pl.BlockSpec((BF, 128), lambda i: (i, 0)),
        pl.BlockSpec((128, 128), lambda i: (0, 0)),
    ],
    out_specs=pl.BlockSpec((BF, 128), lambda i: (i, 0)),
    out_shape=jax.ShapeDtypeStruct((NTF, 128), jnp.float32),
)


def _mid_body(agg_ref, ns2e_ref, nse_ref, wp_ref, bt_ref, g_ref):
    a = agg_ref[0] + agg_ref[1]
    wp = wp_ref[...]
    cvec = jnp.dot(bt_ref[...], wp, preferred_element_type=jnp.float32)
    g_ref[...] = (jnp.dot(a * ns2e_ref[...], wp,
                          preferred_element_type=jnp.float32)
                  + nse_ref[...] * cvec)


_mid_call = pl.pallas_call(
    _mid_body,
    grid=(NTF // BF,),
    in_specs=[
        pl.BlockSpec((2, BF, 128), lambda i: (0, i, 0)),
        pl.BlockSpec((BF, 128), lambda i: (i, 0)),
        pl.BlockSpec((BF, 128), lambda i: (i, 0)),
        pl.BlockSpec((128, 128), lambda i: (0, 0)),
        pl.BlockSpec((1, 128), lambda i: (0, 0)),
    ],
    out_specs=pl.BlockSpec((BF, 128), lambda i: (i, 0)),
    out_shape=jax.ShapeDtypeStruct((NTF, 128), jnp.float32),
)


def _final_body(agg_ref, nde_ref, bt_ref, out_ref):
    h = (agg_ref[0] + agg_ref[1]) * nde_ref[...] + bt_ref[...]
    out_ref[...] = jnp.abs(h)


_final_call = pl.pallas_call(
    _final_body,
    grid=(NTF // BF,),
    in_specs=[
        pl.BlockSpec((2, BF, 128), lambda i: (0, i, 0)),
        pl.BlockSpec((BF, 128), lambda i: (i, 0)),
        pl.BlockSpec((1, 128), lambda i: (0, 0)),
    ],
    out_specs=pl.BlockSpec((BF, 128), lambda i: (i, 0)),
    out_shape=jax.ShapeDtypeStruct((NTF, 128), jnp.float32),
)


def _expand(v):
    return jnp.reshape(
        jnp.broadcast_to(jnp.reshape(v, (NT, 1)), (NT, D)), (NTF, 128))


def kernel(x, edge_index, W0, b0, W1, b1, W2, b2, W3, b3):
    src = edge_index[0]
    dst = edge_index[1]
    pad = EPAD - E
    padv = jnp.full((pad,), N, jnp.int32)
    srcp = jnp.concatenate([src, padv])
    dstp = jnp.concatenate([dst, padv])
    xf = jnp.reshape(jnp.zeros((NT, D), jnp.float32).at[:N].set(x),
                     (NTF, 128))
    eye8 = jnp.eye(8, dtype=jnp.float32)

    degs = _deg_kernel(srcp, dstp)
    degf = jnp.reshape(degs, (4 * NDF, 128))
    ns, nd, ns2 = _norms_call(degf, degf, degf, degf)
    nse_f = _expand(ns)
    ns2e_f = _expand(ns2)
    nde_f = _expand(nd)

    g = _g0_call(xf, nse_f, jnp.kron(eye8, W0))
    for (w, b) in ((W1, b0), (W2, b1), (W3, b2)):
        agg = _seg_kernel(jnp.reshape(g, (NT, D)), srcp, dstp)
        g = _mid_call(jnp.reshape(agg, (NC, NTF, 128)), ns2e_f, nse_f,
                      jnp.kron(eye8, w), jnp.tile(b, 8).reshape(1, 128))
    agg = _seg_kernel(jnp.reshape(g, (NT, D)), srcp, dstp)
    out = _final_call(jnp.reshape(agg, (NC, NTF, 128)), nde_f,
                      jnp.tile(b3, 8).reshape(1, 128))
    return jnp.reshape(out, (NT, D))[:N]


# speedup vs baseline: 1.0754x; 1.0754x over previous
import functools

import jax
import jax.numpy as jnp
from jax import lax
from jax.experimental import pallas as pl
from jax.experimental.pallas import tpu as pltpu
from jax.experimental.pallas import tpu_sc as plsc

N = 100000
E = 3200000
D = 16
NT = 100352
NTF = NT // 8
NC, NS = 2, 16
LANE = 128
CB = 4
RPT = 784
RPTD = RPT
RT = NC * NS * RPT
EPAD = RT * LANE
SLICE = NT // NS
CBL = CB * LANE
NS0 = 54
NS1 = 44
EC0 = NS0 * 4 * CBL
EC1 = NS1 * 4 * CBL

_mesh = plsc.VectorSubcoreMesh(
    core_axis_name="c", subcore_axis_name="s", num_cores=NC, num_subcores=NS
)


@functools.partial(
    pl.kernel,
    out_type=jax.ShapeDtypeStruct((NC, 2, NT), jnp.float32),
    mesh=_mesh,
    compiler_params=pltpu.CompilerParams(use_tc_tiling_on_sc=False),
    scratch_types=[
        pltpu.VMEM((4, CBL), jnp.int32),
        pltpu.VMEM((4, CBL), jnp.int32),
        pltpu.VMEM((CBL,), jnp.float32),
        pltpu.VMEM((SLICE,), jnp.float32),
        pltpu.VMEM_SHARED((NT,), jnp.float32),
        pltpu.VMEM_SHARED((NT,), jnp.float32),
        pltpu.SemaphoreType.DMA,
        pltpu.SemaphoreType.DMA,
        pltpu.SemaphoreType.DMA,
        pltpu.SemaphoreType.DMA,
        pltpu.SemaphoreType.DMA,
        pltpu.SemaphoreType.DMA,
        pltpu.SemaphoreType.DMA,
        pltpu.SemaphoreType.DMA,
    ],
)
def _deg_kernel(src_hbm, dst_hbm, out_hbm, src_v, dst_v, ones_v, z_v,
                dego_s, degi_s, is0, is1, is2, is3, ss0, ss1, ss2, ss3):
    c = lax.axis_index("c")
    s = lax.axis_index("s")
    wid = c * NS + s
    isem = (is0, is1, is2, is3)
    ssem = (ss0, ss1, ss2, ss3)

    for i in range(CBL // 16):
        ones_v[pl.ds(i * 16, 16)] = jnp.ones((16,), jnp.float32)

    def zfill(i, _):
        z_v[pl.ds(i * 16, 16)] = jnp.zeros((16,), jnp.float32)
        return 0

    lax.fori_loop(0, SLICE // 16, zfill, 0)
    pltpu.sync_copy(z_v, dego_s.at[pl.ds(s * SLICE, SLICE)])
    pltpu.sync_copy(z_v, degi_s.at[pl.ds(s * SLICE, SLICE)])
    plsc.subcore_barrier()

    base = jnp.where(c == 0, s * EC0, NS * EC0 + s * EC1)
    nsup = jnp.where(c == 0, NS0, NS1)
    nit = 4 * nsup

    def load_idx(slot, off):
        pltpu.async_copy(src_hbm.at[pl.ds(off, CBL)], src_v.at[slot],
                         isem[slot])
        pltpu.async_copy(dst_hbm.at[pl.ds(off, CBL)], dst_v.at[slot],
                         isem[slot])

    def wait_idx(slot):
        pltpu.make_async_copy(src_hbm.at[pl.ds(base, CBL)], src_v.at[slot],
                              isem[slot]).wait()
        pltpu.make_async_copy(dst_hbm.at[pl.ds(base, CBL)], dst_v.at[slot],
                              isem[slot]).wait()

    def fire_scatters(slot):
        pltpu.async_copy(ones_v, dego_s.at[src_v.at[slot]],
                         ssem[slot], add=True)
        pltpu.async_copy(ones_v, degi_s.at[dst_v.at[slot]],
                         ssem[slot], add=True)

    def wait_scatters(slot):
        pltpu.make_async_copy(ones_v, dego_s.at[src_v.at[slot]],
                              ssem[slot]).wait()
        pltpu.make_async_copy(ones_v, degi_s.at[dst_v.at[slot]],
                              ssem[slot]).wait()

    load_idx(0, base)
    load_idx(1, base + CBL)

    def super_body(sp, _):
        for b in range(4):
            s2 = (b + 2) % 4
            it = 4 * sp + b
            wait_idx(b)
            fire_scatters(b)
            if b < 2:
                @pl.when(jnp.logical_and(it + 2 < nit, sp > 0))
                def _():
                    wait_scatters(s2)
                    load_idx(s2, base + (it + 2) * CBL)

                @pl.when(jnp.logical_and(it + 2 < nit, sp == 0))
                def _():
                    load_idx(s2, base + (it + 2) * CBL)
            else:
                @pl.when(it + 2 < nit)
                def _():
                    wait_scatters(s2)
                    load_idx(s2, base + (it + 2) * CBL)
        return 0

    lax.fori_loop(0, nsup, super_body, 0)

    for b in range(4):
        wait_scatters(b)

    plsc.subcore_barrier()
    pltpu.sync_copy(dego_s.at[pl.ds(s * SLICE, SLICE)],
                    out_hbm.at[c, 0, pl.ds(s * SLICE, SLICE)])
    pltpu.sync_copy(degi_s.at[pl.ds(s * SLICE, SLICE)],
                    out_hbm.at[c, 1, pl.ds(s * SLICE, SLICE)])


@functools.partial(
    pl.kernel,
    out_type=jax.ShapeDtypeStruct((NC, NT, D), jnp.float32),
    mesh=_mesh,
    compiler_params=pltpu.CompilerParams(use_tc_tiling_on_sc=False),
    scratch_types=[
        pltpu.VMEM((4, CBL), jnp.int32),
        pltpu.VMEM((4, CBL), jnp.int32),
        pltpu.VMEM((2, CBL, D), jnp.float32),
        pltpu.VMEM((LANE, D), jnp.float32),
        pltpu.VMEM_SHARED((NT, D), jnp.float32),
        pltpu.SemaphoreType.DMA,
        pltpu.SemaphoreType.DMA,
        pltpu.SemaphoreType.DMA,
        pltpu.SemaphoreType.DMA,
        pltpu.SemaphoreType.DMA,
        pltpu.SemaphoreType.DMA,
        pltpu.SemaphoreType.DMA,
        pltpu.SemaphoreType.DMA,
    ],
)
def _seg_kernel(g_hbm, src_hbm, dst_hbm, out_hbm, src_v, dst_v, rows_v,
                zrow_v, acc_s, gsem0, gsem1, ssem0, ssem1,
                isem0, isem1, isem2, isem3):
    c = lax.axis_index("c")
    s = lax.axis_index("s")
    wid = c * NS + s
    gsem = (gsem0, gsem1)
    ssem = (ssem0, ssem1)
    isem = (isem0, isem1, isem2, isem3)

    for i in range(LANE):
        zrow_v[i, :] = jnp.zeros((D,), jnp.float32)

    def zfill(k, _):
        pltpu.async_copy(zrow_v,
                         acc_s.at[pl.ds(s * SLICE + k * LANE, LANE)],
                         gsem0)
        return 0

    lax.fori_loop(0, SLICE // LANE, zfill, 0)

    def zdrain(k, _):
        pltpu.make_async_copy(
            zrow_v, acc_s.at[pl.ds(s * SLICE + k * LANE, LANE)],
            gsem0).wait()
        return 0

    lax.fori_loop(0, SLICE // LANE, zdrain, 0)
    plsc.subcore_barrier()

    base = jnp.where(c == 0, s * EC0, NS * EC0 + s * EC1)
    nsup = jnp.where(c == 0, NS0, NS1)

    def load_idx(slot, off):
        pltpu.async_copy(src_hbm.at[pl.ds(off, CBL)], src_v.at[slot],
                         isem[slot])
        pltpu.async_copy(dst_hbm.at[pl.ds(off, CBL)], dst_v.at[slot],
                         isem[slot])

    def wait_idx(slot):
        pltpu.make_async_copy(src_hbm.at[pl.ds(base, CBL)], src_v.at[slot],
                              isem[slot]).wait()
        pltpu.make_async_copy(dst_hbm.at[pl.ds(base, CBL)], dst_v.at[slot],
                              isem[slot]).wait()

    def fire_gathers(rs, qs):
        pltpu.async_copy(g_hbm.at[src_v.at[qs]], rows_v.at[rs], gsem[rs])

    def wait_gathers(rs, qs):
        pltpu.make_async_copy(g_hbm.at[src_v.at[qs]], rows_v.at[rs],
                              gsem[rs]).wait()

    def fire_scatters(rs, qs):
        pltpu.async_copy(rows_v.at[rs], acc_s.at[dst_v.at[qs]], ssem[rs],
                         add=True)

    def wait_scatters(rs, qs):
        pltpu.make_async_copy(rows_v.at[rs], acc_s.at[dst_v.at[qs]],
                              ssem[rs]).wait()

    load_idx(0, base)
    load_idx(1, base + CBL)

    def super_body(sp, _):
        for u in range(4):
            b = u % 2
            q = u
            q2 = (u + 2) % 4
            ob = 1 - b
            oq = (u + 3) % 4
            j = 4 * sp + u
            if u < 2:
                @pl.when(sp > 0)
                def _():
                    wait_scatters(b, q2)
            else:
                wait_scatters(b, q2)
            if u < 2:
                load_idx(q2, base + (j + 2) * CBL)
            else:
                @pl.when(sp < nsup - 1)
                def _():
                    load_idx(q2, base + (j + 2) * CBL)
            wait_idx(q)
            fire_gathers(b, q)
            if u == 0:
                @pl.when(sp > 0)
                def _():
                    wait_gathers(ob, oq)
                    fire_scatters(ob, oq)
            else:
                wait_gathers(ob, oq)
                fire_scatters(ob, oq)
        return 0

    lax.fori_loop(0, nsup, super_body, 0)

    wait_gathers(1, 3)
    fire_scatters(1, 3)
    wait_scatters(0, 2)
    wait_scatters(1, 3)

    plsc.subcore_barrier()
    pltpu.sync_copy(acc_s.at[pl.ds(s * SLICE, SLICE)],
                    out_hbm.at[c, pl.ds(s * SLICE, SLICE)])


NDF = NT // 128
DBLK = 112
BF = 1568


def _norms_body(d00_ref, d01_ref, d10_ref, d11_ref, ns_ref, nd_ref, ns2_ref):
    i = pl.program_id(0)
    dego = d00_ref[...] + d10_ref[...]
    degi = d01_ref[...] + d11_ref[...]
    ns = lax.rsqrt(jnp.maximum(dego, 1.0))
    nd = lax.rsqrt(jnp.maximum(degi, 1.0))
    node = (i * (DBLK * 128)
            + lax.broadcasted_iota(jnp.int32, (DBLK, 128), 0) * 128
            + lax.broadcasted_iota(jnp.int32, (DBLK, 128), 1))
    valid = node < N
    ns_ref[...] = jnp.where(valid, ns, 0.0)
    nd_ref[...] = jnp.where(valid, nd, 0.0)
    ns2_ref[...] = jnp.where(valid, ns * nd, 0.0)


_NDG = NDF // DBLK

_norms_call = pl.pallas_call(
    _norms_body,
    grid=(_NDG,),
    in_specs=[
        pl.BlockSpec((DBLK, 128), lambda i: (i, 0)),
        pl.BlockSpec((DBLK, 128), lambda i: (_NDG + i, 0)),
        pl.BlockSpec((DBLK, 128), lambda i: (2 * _NDG + i, 0)),
        pl.BlockSpec((DBLK, 128), lambda i: (3 * _NDG + i, 0)),
    ],
    out_specs=[pl.BlockSpec((DBLK, 128), lambda i: (i, 0))] * 3,
    out_shape=[jax.ShapeDtypeStruct((NDF, 128), jnp.float32)] * 3,
)


def _g0_body(x_ref, nse_ref, wp_ref, g_ref):
    g_ref[...] = jnp.dot(x_ref[...] * nse_ref[...], wp_ref[...],
                         preferred_element_type=jnp.float32)


_g0_call = pl.pallas_call(
    _g0_body,
    grid=(NTF // BF,),
    in_specs=[
        pl.BlockSpec((BF, 128), lambda i: (i, 0)),
        pl.BlockSpec((BF, 128), lambda i: (i, 0)),
        pl.BlockSpec((128, 128), lambda i: (0, 0)),
    ],
    out_specs=pl.BlockSpec((BF, 128), lambda i: (i, 0)),
    out_shape=jax.ShapeDtypeStruct((NTF, 128), jnp.float32),
)


def _mid_body(agg_ref, ns2e_ref, nse_ref, wp_ref, bt_ref, g_ref):
    a = agg_ref[0] + agg_ref[1]
    wp = wp_ref[...]
    cvec = jnp.dot(bt_ref[...], wp, preferred_element_type=jnp.float32)
    g_ref[...] = (jnp.dot(a * ns2e_ref[...], wp,
                          preferred_element_type=jnp.float32)
                  + nse_ref[...] * cvec)


_mid_call = pl.pallas_call(
    _mid_body,
    grid=(NTF // BF,),
    in_specs=[
        pl.BlockSpec((2, BF, 128), lambda i: (0, i, 0)),
        pl.BlockSpec((BF, 128), lambda i: (i, 0)),
        pl.BlockSpec((BF, 128), lambda i: (i, 0)),
        pl.BlockSpec((128, 128), lambda i: (0, 0)),
        pl.BlockSpec((1, 128), lambda i: (0, 0)),
    ],
    out_specs=pl.BlockSpec((BF, 128), lambda i: (i, 0)),
    out_shape=jax.ShapeDtypeStruct((NTF, 128), jnp.float32),
)


def _final_body(agg_ref, nde_ref, bt_ref, out_ref):
    h = (agg_ref[0] + agg_ref[1]) * nde_ref[...] + bt_ref[...]
    out_ref[...] = jnp.abs(h)


_final_call = pl.pallas_call(
    _final_body,
    grid=(NTF // BF,),
    in_specs=[
        pl.BlockSpec((2, BF, 128), lambda i: (0, i, 0)),
        pl.BlockSpec((BF, 128), lambda i: (i, 0)),
        pl.BlockSpec((1, 128), lambda i: (0, 0)),
    ],
    out_specs=pl.BlockSpec((BF, 128), lambda i: (i, 0)),
    out_shape=jax.ShapeDtypeStruct((NTF, 128), jnp.float32),
)


def _expand(v):
    return jnp.reshape(
        jnp.broadcast_to(jnp.reshape(v, (NT, 1)), (NT, D)), (NTF, 128))


def kernel(x, edge_index, W0, b0, W1, b1, W2, b2, W3, b3):
    src = edge_index[0]
    dst = edge_index[1]
    pad = EPAD - E
    padv = jnp.full((pad,), N, jnp.int32)
    srcp = jnp.concatenate([src, padv])
    dstp = jnp.concatenate([dst, padv])
    xf = jnp.reshape(jnp.zeros((NT, D), jnp.float32).at[:N].set(x),
                     (NTF, 128))
    eye8 = jnp.eye(8, dtype=jnp.float32)

    degs = _deg_kernel(srcp, dstp)
    degf = jnp.reshape(degs, (4 * NDF, 128))
    ns, nd, ns2 = _norms_call(degf, degf, degf, degf)
    nse_f = _expand(ns)
    ns2e_f = _expand(ns2)
    nde_f = _expand(nd)

    g = _g0_call(xf, nse_f, jnp.kron(eye8, W0))
    for (w, b) in ((W1, b0), (W2, b1), (W3, b2)):
        agg = _seg_kernel(jnp.reshape(g, (NT, D)), srcp, dstp)
        g = _mid_call(jnp.reshape(agg, (NC, NTF, 128)), ns2e_f, nse_f,
                      jnp.kron(eye8, w), jnp.tile(b, 8).reshape(1, 128))
    agg = _seg_kernel(jnp.reshape(g, (NT, D)), srcp, dstp)
    out = _final_call(jnp.reshape(agg, (NC, NTF, 128)), nde_f,
                      jnp.tile(b3, 8).reshape(1, 128))
    return jnp.reshape(out, (NT, D))[:N]


# speedup vs baseline: 1.0873x; 1.0111x over previous
import functools

import jax
import jax.numpy as jnp
from jax import lax
from jax.experimental import pallas as pl
from jax.experimental.pallas import tpu as pltpu
from jax.experimental.pallas import tpu_sc as plsc

N = 100000
E = 3200000
D = 16
NT = 100352
NTF = NT // 8
NC, NS = 2, 16
LANE = 128
CB = 4
RPT = 784
RPTD = RPT
RT = NC * NS * RPT
EPAD = RT * LANE
SLICE = NT // NS
CBL = CB * LANE
NS0 = 55
NS1 = 43
EC0 = NS0 * 4 * CBL
EC1 = NS1 * 4 * CBL

_mesh = plsc.VectorSubcoreMesh(
    core_axis_name="c", subcore_axis_name="s", num_cores=NC, num_subcores=NS
)


@functools.partial(
    pl.kernel,
    out_type=jax.ShapeDtypeStruct((NC, 2, NT), jnp.float32),
    mesh=_mesh,
    compiler_params=pltpu.CompilerParams(use_tc_tiling_on_sc=False),
    scratch_types=[
        pltpu.VMEM((4, CBL), jnp.int32),
        pltpu.VMEM((4, CBL), jnp.int32),
        pltpu.VMEM((CBL,), jnp.float32),
        pltpu.VMEM((SLICE,), jnp.float32),
        pltpu.VMEM_SHARED((NT,), jnp.float32),
        pltpu.VMEM_SHARED((NT,), jnp.float32),
        pltpu.SemaphoreType.DMA,
        pltpu.SemaphoreType.DMA,
        pltpu.SemaphoreType.DMA,
        pltpu.SemaphoreType.DMA,
        pltpu.SemaphoreType.DMA,
        pltpu.SemaphoreType.DMA,
        pltpu.SemaphoreType.DMA,
        pltpu.SemaphoreType.DMA,
    ],
)
def _deg_kernel(src_hbm, dst_hbm, out_hbm, src_v, dst_v, ones_v, z_v,
                dego_s, degi_s, is0, is1, is2, is3, ss0, ss1, ss2, ss3):
    c = lax.axis_index("c")
    s = lax.axis_index("s")
    wid = c * NS + s
    isem = (is0, is1, is2, is3)
    ssem = (ss0, ss1, ss2, ss3)

    for i in range(CBL // 16):
        ones_v[pl.ds(i * 16, 16)] = jnp.ones((16,), jnp.float32)

    def zfill(i, _):
        z_v[pl.ds(i * 16, 16)] = jnp.zeros((16,), jnp.float32)
        return 0

    lax.fori_loop(0, SLICE // 16, zfill, 0)
    pltpu.sync_copy(z_v, dego_s.at[pl.ds(s * SLICE, SLICE)])
    pltpu.sync_copy(z_v, degi_s.at[pl.ds(s * SLICE, SLICE)])
    plsc.subcore_barrier()

    base = jnp.where(c == 0, s * EC0, NS * EC0 + s * EC1)
    nsup = jnp.where(c == 0, NS0, NS1)
    nit = 4 * nsup

    def load_idx(slot, off):
        pltpu.async_copy(src_hbm.at[pl.ds(off, CBL)], src_v.at[slot],
                         isem[slot])
        pltpu.async_copy(dst_hbm.at[pl.ds(off, CBL)], dst_v.at[slot],
                         isem[slot])

    def wait_idx(slot):
        pltpu.make_async_copy(src_hbm.at[pl.ds(base, CBL)], src_v.at[slot],
                              isem[slot]).wait()
        pltpu.make_async_copy(dst_hbm.at[pl.ds(base, CBL)], dst_v.at[slot],
                              isem[slot]).wait()

    def fire_scatters(slot):
        pltpu.async_copy(ones_v, dego_s.at[src_v.at[slot]],
                         ssem[slot], add=True)
        pltpu.async_copy(ones_v, degi_s.at[dst_v.at[slot]],
                         ssem[slot], add=True)

    def wait_scatters(slot):
        pltpu.make_async_copy(ones_v, dego_s.at[src_v.at[slot]],
                              ssem[slot]).wait()
        pltpu.make_async_copy(ones_v, degi_s.at[dst_v.at[slot]],
                              ssem[slot]).wait()

    load_idx(0, base)
    load_idx(1, base + CBL)

    def super_body(sp, _):
        for b in range(4):
            s2 = (b + 2) % 4
            it = 4 * sp + b
            wait_idx(b)
            fire_scatters(b)
            if b < 2:
                @pl.when(jnp.logical_and(it + 2 < nit, sp > 0))
                def _():
                    wait_scatters(s2)
                    load_idx(s2, base + (it + 2) * CBL)

                @pl.when(jnp.logical_and(it + 2 < nit, sp == 0))
                def _():
                    load_idx(s2, base + (it + 2) * CBL)
            else:
                @pl.when(it + 2 < nit)
                def _():
                    wait_scatters(s2)
                    load_idx(s2, base + (it + 2) * CBL)
        return 0

    lax.fori_loop(0, nsup, super_body, 0)

    for b in range(4):
        wait_scatters(b)

    plsc.subcore_barrier()
    pltpu.sync_copy(dego_s.at[pl.ds(s * SLICE, SLICE)],
                    out_hbm.at[c, 0, pl.ds(s * SLICE, SLICE)])
    pltpu.sync_copy(degi_s.at[pl.ds(s * SLICE, SLICE)],
                    out_hbm.at[c, 1, pl.ds(s * SLICE, SLICE)])


@functools.partial(
    pl.kernel,
    out_type=jax.ShapeDtypeStruct((NC, NT, D), jnp.float32),
    mesh=_mesh,
    compiler_params=pltpu.CompilerParams(use_tc_tiling_on_sc=False),
    scratch_types=[
        pltpu.VMEM((4, CBL), jnp.int32),
        pltpu.VMEM((4, CBL), jnp.int32),
        pltpu.VMEM((2, CBL, D), jnp.float32),
        pltpu.VMEM((LANE, D), jnp.float32),
        pltpu.VMEM_SHARED((NT, D), jnp.float32),
        pltpu.SemaphoreType.DMA,
        pltpu.SemaphoreType.DMA,
        pltpu.SemaphoreType.DMA,
        pltpu.SemaphoreType.DMA,
        pltpu.SemaphoreType.DMA,
        pltpu.SemaphoreType.DMA,
        pltpu.SemaphoreType.DMA,
        pltpu.SemaphoreType.DMA,
    ],
)
def _seg_kernel(g_hbm, src_hbm, dst_hbm, out_hbm, src_v, dst_v, rows_v,
                zrow_v, acc_s, gsem0, gsem1, ssem0, ssem1,
                isem0, isem1, isem2, isem3):
    c = lax.axis_index("c")
    s = lax.axis_index("s")
    wid = c * NS + s
    gsem = (gsem0, gsem1)
    ssem = (ssem0, ssem1)
    isem = (isem0, isem1, isem2, isem3)

    for i in range(LANE):
        zrow_v[i, :] = jnp.zeros((D,), jnp.float32)

    def zfill(k, _):
        pltpu.async_copy(zrow_v,
                         acc_s.at[pl.ds(s * SLICE + k * LANE, LANE)],
                         gsem0)
        return 0

    lax.fori_loop(0, SLICE // LANE, zfill, 0)

    def zdrain(k, _):
        pltpu.make_async_copy(
            zrow_v, acc_s.at[pl.ds(s * SLICE + k * LANE, LANE)],
            gsem0).wait()
        return 0

    lax.fori_loop(0, SLICE // LANE, zdrain, 0)
    plsc.subcore_barrier()

    base = jnp.where(c == 0, s * EC0, NS * EC0 + s * EC1)
    nsup = jnp.where(c == 0, NS0, NS1)

    def load_idx(slot, off):
        pltpu.async_copy(src_hbm.at[pl.ds(off, CBL)], src_v.at[slot],
                         isem[slot])
        pltpu.async_copy(dst_hbm.at[pl.ds(off, CBL)], dst_v.at[slot],
                         isem[slot])

    def wait_idx(slot):
        pltpu.make_async_copy(src_hbm.at[pl.ds(base, CBL)], src_v.at[slot],
                              isem[slot]).wait()
        pltpu.make_async_copy(dst_hbm.at[pl.ds(base, CBL)], dst_v.at[slot],
                              isem[slot]).wait()

    def fire_gathers(rs, qs):
        pltpu.async_copy(g_hbm.at[src_v.at[qs]], rows_v.at[rs], gsem[rs])

    def wait_gathers(rs, qs):
        pltpu.make_async_copy(g_hbm.at[src_v.at[qs]], rows_v.at[rs],
                              gsem[rs]).wait()

    def fire_scatters(rs, qs):
        pltpu.async_copy(rows_v.at[rs], acc_s.at[dst_v.at[qs]], ssem[rs],
                         add=True)

    def wait_scatters(rs, qs):
        pltpu.make_async_copy(rows_v.at[rs], acc_s.at[dst_v.at[qs]],
                              ssem[rs]).wait()

    load_idx(0, base)
    load_idx(1, base + CBL)

    def super_body(sp, _):
        for u in range(4):
            b = u % 2
            q = u
            q2 = (u + 2) % 4
            ob = 1 - b
            oq = (u + 3) % 4
            j = 4 * sp + u
            if u < 2:
                @pl.when(sp > 0)
                def _():
                    wait_scatters(b, q2)
            else:
                wait_scatters(b, q2)
            if u < 2:
                load_idx(q2, base + (j + 2) * CBL)
            else:
                @pl.when(sp < nsup - 1)
                def _():
                    load_idx(q2, base + (j + 2) * CBL)
            wait_idx(q)
            fire_gathers(b, q)
            if u == 0:
                @pl.when(sp > 0)
                def _():
                    wait_gathers(ob, oq)
                    fire_scatters(ob, oq)
            else:
                wait_gathers(ob, oq)
                fire_scatters(ob, oq)
        return 0

    lax.fori_loop(0, nsup, super_body, 0)

    wait_gathers(1, 3)
    fire_scatters(1, 3)
    wait_scatters(0, 2)
    wait_scatters(1, 3)

    plsc.subcore_barrier()
    pltpu.sync_copy(acc_s.at[pl.ds(s * SLICE, SLICE)],
                    out_hbm.at[c, pl.ds(s * SLICE, SLICE)])


NDF = NT // 128
DBLK = 112
BF = 1568


def _norms_body(d00_ref, d01_ref, d10_ref, d11_ref, ns_ref, nd_ref, ns2_ref):
    i = pl.program_id(0)
    dego = d00_ref[...] + d10_ref[...]
    degi = d01_ref[...] + d11_ref[...]
    ns = lax.rsqrt(jnp.maximum(dego, 1.0))
    nd = lax.rsqrt(jnp.maximum(degi, 1.0))
    node = (i * (DBLK * 128)
            + lax.broadcasted_iota(jnp.int32, (DBLK, 128), 0) * 128
            + lax.broadcasted_iota(jnp.int32, (DBLK, 128), 1))
    valid = node < N
    ns_ref[...] = jnp.where(valid, ns, 0.0)
    nd_ref[...] = jnp.where(valid, nd, 0.0)
    ns2_ref[...] = jnp.where(valid, ns * nd, 0.0)


_NDG = NDF // DBLK

_norms_call = pl.pallas_call(
    _norms_body,
    grid=(_NDG,),
    in_specs=[
        pl.BlockSpec((DBLK, 128), lambda i: (i, 0)),
        pl.BlockSpec((DBLK, 128), lambda i: (_NDG + i, 0)),
        pl.BlockSpec((DBLK, 128), lambda i: (2 * _NDG + i, 0)),
        pl.BlockSpec((DBLK, 128), lambda i: (3 * _NDG + i, 0)),
    ],
    out_specs=[pl.BlockSpec((DBLK, 128), lambda i: (i, 0))] * 3,
    out_shape=[jax.ShapeDtypeStruct((NDF, 128), jnp.float32)] * 3,
)


def _g0_body(x_ref, nse_ref, wp_ref, g_ref):
    g_ref[...] = jnp.dot(x_ref[...] * nse_ref[...], wp_ref[...],
                         preferred_element_type=jnp.float32)


_g0_call = pl.pallas_call(
    _g0_body,
    grid=(NTF // BF,),
    in_specs=[
        pl.BlockSpec((BF, 128), lambda i: (i, 0)),
        pl.BlockSpec((BF, 128), lambda i: (i, 0)),
        pl.BlockSpec((128, 128), lambda i: (0, 0)),
    ],
    out_specs=pl.BlockSpec((BF, 128), lambda i: (i, 0)),
    out_shape=jax.ShapeDtypeStruct((NTF, 128), jnp.float32),
)


def _mid_body(agg_ref, ns2e_ref, nse_ref, wp_ref, bt_ref, g_ref):
    a = agg_ref[0] + agg_ref[1]
    wp = wp_ref[...]
    cvec = jnp.dot(bt_ref[...], wp, preferred_element_type=jnp.float32)
    g_ref[...] = (jnp.dot(a * ns2e_ref[...], wp,
                          preferred_element_type=jnp.float32)
                  + nse_ref[...] * cvec)


_mid_call = pl.pallas_call(
    _mid_body,
    grid=(NTF // BF,),
    in_specs=[
        pl.BlockSpec((2, BF, 128), lambda i: (0, i, 0)),
        pl.BlockSpec((BF, 128), lambda i: (i, 0)),
        pl.BlockSpec((BF, 128), lambda i: (i, 0)),
        pl.BlockSpec((128, 128), lambda i: (0, 0)),
        pl.BlockSpec((1, 128), lambda i: (0, 0)),
    ],
    out_specs=pl.BlockSpec((BF, 128), lambda i: (i, 0)),
    out_shape=jax.ShapeDtypeStruct((NTF, 128), jnp.float32),
)


def _final_body(agg_ref, nde_ref, bt_ref, out_ref):
    h = (agg_ref[0] + agg_ref[1]) * nde_ref[...] + bt_ref[...]
    out_ref[...] = jnp.abs(h)


_final_call = pl.pallas_call(
    _final_body,
    grid=(NTF // BF,),
    in_specs=[
        pl.BlockSpec((2, BF, 128), lambda i: (0, i, 0)),
        pl.BlockSpec((BF, 128), lambda i: (i, 0)),
        pl.BlockSpec((1, 128), lambda i: (0, 0)),
    ],
    out_specs=pl.BlockSpec((BF, 128), lambda i: (i, 0)),
    out_shape=jax.ShapeDtypeStruct((NTF, 128), jnp.float32),
)


def _expand(v):
    return jnp.reshape(
        jnp.broadcast_to(jnp.reshape(v, (NT, 1)), (NT, D)), (NTF, 128))


def kernel(x, edge_index, W0, b0, W1, b1, W2, b2, W3, b3):
    src = edge_index[0]
    dst = edge_index[1]
    pad = EPAD - E
    padv = jnp.full((pad,), N, jnp.int32)
    srcp = jnp.concatenate([src, padv])
    dstp = jnp.concatenate([dst, padv])
    xf = jnp.reshape(jnp.zeros((NT, D), jnp.float32).at[:N].set(x),
                     (NTF, 128))
    eye8 = jnp.eye(8, dtype=jnp.float32)

    degs = _deg_kernel(srcp, dstp)
    degf = jnp.reshape(degs, (4 * NDF, 128))
    ns, nd, ns2 = _norms_call(degf, degf, degf, degf)
    nse_f = _expand(ns)
    ns2e_f = _expand(ns2)
    nde_f = _expand(nd)

    g = _g0_call(xf, nse_f, jnp.kron(eye8, W0))
    for (w, b) in ((W1, b0), (W2, b1), (W3, b2)):
        agg = _seg_kernel(jnp.reshape(g, (NT, D)), srcp, dstp)
        g = _mid_call(jnp.reshape(agg, (NC, NTF, 128)), ns2e_f, nse_f,
                      jnp.kron(eye8, w), jnp.tile(b, 8).reshape(1, 128))
    agg = _seg_kernel(jnp.reshape(g, (NT, D)), srcp, dstp)
    out = _final_call(jnp.reshape(agg, (NC, NTF, 128)), nde_f,
                      jnp.tile(b3, 8).reshape(1, 128))
    return jnp.reshape(out, (NT, D))[:N]
